# Initial kernel scaffold; baseline (speedup 1.0000x reference)
import jax
import jax.numpy as jnp
from jax.experimental import pallas as pl


def kernel(x, edge_index, batch, W_in, b_in, conv_W1, conv_b1, conv_W2, conv_b2, W_fc, b_fc):
    raise NotImplementedError("write your pallas kernel here")


# speedup vs baseline: 8.2370x; 8.2370x over previous
import functools

import jax
import jax.numpy as jnp
from jax import lax
from jax.experimental import pallas as pl
from jax.experimental.pallas import tpu as pltpu
from jax.experimental.pallas import tpu_sc as plsc

N = 10000
E = 320000
D = 128
H = 128
L = 128
NL = 3
G = 64

NC = 2
NS = 16
NW = NC * NS
CH = 128
NCHUNK = 80
E_PAD = NW * NCHUNK * CH
N_PAD = 10240
ROW_BLK = 1000


def _sc_aggregate(h, src3, dst3, zeros_blk):
    mesh = plsc.VectorSubcoreMesh(
        core_axis_name="c", subcore_axis_name="s",
        num_cores=NC, num_subcores=NS)

    @functools.partial(
        pl.kernel,
        out_type=jax.ShapeDtypeStruct((NC, N_PAD, H), jnp.float32),
        mesh=mesh,
        scratch_types=[
            pltpu.VMEM((NCHUNK, CH), jnp.int32),
            pltpu.VMEM((NCHUNK, CH), jnp.int32),
            pltpu.VMEM((CH, H), jnp.float32),
            pltpu.VMEM_SHARED((N_PAD, H), jnp.float32),
            pltpu.SemaphoreType.DMA,
        ],
    )
    def k(h_hbm, src_hbm, dst_hbm, z_hbm, out_hbm, src_v, dst_v, buf, agg_sh, sem):
        cid = lax.axis_index("c")
        sid = lax.axis_index("s")
        wid = cid * NS + sid

        pltpu.sync_copy(z_hbm, buf)
        for b in range(N_PAD // CH // NS):
            row0 = sid * (N_PAD // NS) + b * CH
            pltpu.sync_copy(buf, agg_sh.at[pl.ds(row0, CH)])
        plsc.subcore_barrier()

        pltpu.sync_copy(src_hbm.at[wid], src_v)
        pltpu.sync_copy(dst_hbm.at[wid], dst_v)

        def body(j, carry):
            pltpu.async_copy(h_hbm.at[src_v.at[j]], buf, sem).wait()
            pltpu.sync_copy(buf, agg_sh.at[dst_v.at[j]], add=True)
            return carry

        lax.fori_loop(0, NCHUNK, body, 0)
        plsc.subcore_barrier()

        for b in range(N_PAD // CH // NS):
            row0 = sid * (N_PAD // NS) + b * CH
            pltpu.sync_copy(agg_sh.at[pl.ds(row0, CH)], buf)
            pltpu.sync_copy(buf, out_hbm.at[cid, pl.ds(row0, CH)])

    return k(h, src3, dst3, zeros_blk)


def _tc_linear(x, W, b2d):
    n = x.shape[0]

    def body(x_ref, w_ref, b_ref, o_ref):
        o_ref[...] = jnp.dot(x_ref[...], w_ref[...],
                             preferred_element_type=jnp.float32) + b_ref[...]

    return pl.pallas_call(
        body,
        grid=(n // ROW_BLK,),
        in_specs=[
            pl.BlockSpec((ROW_BLK, H), lambda i: (i, 0)),
            pl.BlockSpec((H, H), lambda i: (0, 0)),
            pl.BlockSpec((1, H), lambda i: (0, 0)),
        ],
        out_specs=pl.BlockSpec((ROW_BLK, H), lambda i: (i, 0)),
        out_shape=jax.ShapeDtypeStruct((n, H), jnp.float32),
    )(x, W, b2d)


def _tc_mlp(agg2, h, W1, b1, W2, b2):
    def body(a_ref, h_ref, w1_ref, b1_ref, w2_ref, b2_ref, o_ref):
        a = a_ref[...]
        hb = h_ref[...]
        m = a[0] + a[1] + hb
        m = jnp.maximum(jnp.dot(m, w1_ref[...],
                                preferred_element_type=jnp.float32) + b1_ref[...], 0.0)
        m = jnp.maximum(jnp.dot(m, w2_ref[...],
                                preferred_element_type=jnp.float32) + b2_ref[...], 0.0)
        o_ref[...] = m + hb

    return pl.pallas_call(
        body,
        grid=(N // ROW_BLK,),
        in_specs=[
            pl.BlockSpec((NC, ROW_BLK, H), lambda i: (0, i, 0)),
            pl.BlockSpec((ROW_BLK, H), lambda i: (i, 0)),
            pl.BlockSpec((H, H), lambda i: (0, 0)),
            pl.BlockSpec((1, H), lambda i: (0, 0)),
            pl.BlockSpec((H, H), lambda i: (0, 0)),
            pl.BlockSpec((1, H), lambda i: (0, 0)),
        ],
        out_specs=pl.BlockSpec((ROW_BLK, H), lambda i: (i, 0)),
        out_shape=jax.ShapeDtypeStruct((N, H), jnp.float32),
    )(agg2, h, W1, b1, W2, b2)


def _tc_pool_fc(h, batch3, W_fc, b_fc2d):
    nblk = N // ROW_BLK

    def body(h_ref, bt_ref, w_ref, b_ref, o_ref, acc_ref):
        i = pl.program_id(0)

        @pl.when(i == 0)
        def _():
            acc_ref[...] = jnp.zeros_like(acc_ref)

        seg = bt_ref[0]
        gids = lax.broadcasted_iota(jnp.int32, (G, ROW_BLK), 0)
        onehot = (gids == seg).astype(jnp.float32)
        acc_ref[...] += jnp.dot(onehot, h_ref[...],
                                preferred_element_type=jnp.float32)

        @pl.when(i == nblk - 1)
        def _():
            o_ref[...] = jnp.dot(acc_ref[...], w_ref[...],
                                 preferred_element_type=jnp.float32) + b_ref[...]

    return pl.pallas_call(
        body,
        grid=(nblk,),
        in_specs=[
            pl.BlockSpec((ROW_BLK, H), lambda i: (i, 0)),
            pl.BlockSpec((1, 1, ROW_BLK), lambda i: (i, 0, 0)),
            pl.BlockSpec((H, L), lambda i: (0, 0)),
            pl.BlockSpec((1, L), lambda i: (0, 0)),
        ],
        out_specs=pl.BlockSpec((G, L), lambda i: (0, 0)),
        out_shape=jax.ShapeDtypeStruct((G, L), jnp.float32),
        scratch_shapes=[pltpu.VMEM((G, L), jnp.float32)],
    )(h, batch3, W_fc, b_fc2d)


def kernel(x, edge_index, batch, W_in, b_in, conv_W1, conv_b1, conv_W2, conv_b2, W_fc, b_fc):
    src = edge_index[0].astype(jnp.int32)
    dst = edge_index[1].astype(jnp.int32)
    pad = E_PAD - E
    pad_src = (jnp.arange(pad, dtype=jnp.int32) * 97) % N
    pad_dst = N + (jnp.arange(pad, dtype=jnp.int32) % (N_PAD - N))
    src3 = jnp.concatenate([src, pad_src]).reshape(NW, NCHUNK, CH)
    dst3 = jnp.concatenate([dst, pad_dst]).reshape(NW, NCHUNK, CH)
    zeros_blk = jnp.zeros((CH, H), jnp.float32)
    batch3 = batch.astype(jnp.int32).reshape(N // ROW_BLK, 1, ROW_BLK)

    h = _tc_linear(x, W_in, b_in.reshape(1, H))
    for l in range(NL):
        agg2 = _sc_aggregate(h, src3, dst3, zeros_blk)
        h = _tc_mlp(agg2, h, conv_W1[l], conv_b1[l].reshape(1, H),
                    conv_W2[l], conv_b2[l].reshape(1, H))
    return _tc_pool_fc(h, batch3, W_fc, b_fc.reshape(1, L))
